# Initial kernel scaffold; baseline (speedup 1.0000x reference)
import jax
import jax.numpy as jnp
from jax.experimental import pallas as pl


def kernel(feat0, feat1, feat2, e0W1, e0b1, e0W2, e0b2, e1W1, e1b1, e1W2, e1b2, e2W1, e2b1, e2W2, e2b2, g0W, g0b, g1W, g1b, g2W, g2b, hW1, hb1, hW2, hb2):
    raise NotImplementedError("write your pallas kernel here")


# speedup vs baseline: 18.5980x; 18.5980x over previous
import functools

import jax
import jax.numpy as jnp
from jax import lax
from jax.experimental import pallas as pl
from jax.experimental.pallas import tpu as pltpu

_C = 256
_LEVELS = [(128, 128), (64, 64), (32, 32)]
_CC = 32
_CH = 2048


def _body(H, W, f_ref, w1, b1, w2, b2, gw0, gb0, gw1, gb1, gw2, gb2,
          hw1, hb1, hw2, hb2, out_ref, A, B):
    N = H * W
    ch = min(_CH, N)
    logw = W.bit_length() - 1

    n = lax.broadcasted_iota(jnp.int32, (1, N), 1)
    col = n & (W - 1)
    row = lax.shift_right_logical(n, logw)
    f32 = jnp.float32
    top = (row == 0)
    bot = (row == H - 1)
    lft = (col == 0)
    rgt = (col == W - 1)
    deg = (5.0 - top.astype(f32) - bot.astype(f32)
           - lft.astype(f32) - rgt.astype(f32))
    dinv = lax.rsqrt(deg)
    mU = 1.0 - top.astype(f32)
    mD = 1.0 - bot.astype(f32)
    mL = 1.0 - lft.astype(f32)
    mR = 1.0 - rgt.astype(f32)

    def roll(x, k):
        return pltpu.roll(x, k % N, 1)

    for n0 in range(0, N, ch):
        sl = slice(n0, n0 + ch)
        x1 = jnp.maximum(
            jnp.dot(w1[:], f_ref[:, sl], preferred_element_type=f32) + b1[:],
            0.0)
        A[:, sl] = jnp.dot(w2[:], x1, preferred_element_type=f32) + b2[:]

    layers = ((gw0, gb0, A, B), (gw1, gb1, B, A), (gw2, gb2, A, B))
    for gw, gb, src, dst in layers:
        for n0 in range(0, N, ch):
            sl = slice(n0, n0 + ch)
            src[:, sl] = jnp.dot(gw[:], src[:, sl] * dinv[:, sl],
                                 preferred_element_type=f32)
        for c0 in range(0, _C, _CC):
            cs = slice(c0, c0 + _CC)
            g = src[cs, :]
            agg = (g
                   + mU * roll(g, W) + mD * roll(g, -W)
                   + mL * roll(g, 1) + mR * roll(g, -1))
            dst[cs, :] = dinv * agg + gb[cs, :]

    for n0 in range(0, N, ch):
        sl = slice(n0, n0 + ch)
        t = jnp.maximum(
            jnp.dot(hw1[:], B[:, sl], preferred_element_type=f32) + hb1[:],
            0.0)
        out_ref[:, sl] = jnp.dot(hw2[:], t, preferred_element_type=f32) + hb2[:]


@jax.jit
def kernel(feat0, feat1, feat2, e0W1, e0b1, e0W2, e0b2, e1W1, e1b1, e1W2,
           e1b2, e2W1, e2b1, e2W2, e2b2, g0W, g0b, g1W, g1b, g2W, g2b,
           hW1, hb1, hW2, hb2):
    feats = (feat0, feat1, feat2)
    enc = ((e0W1, e0b1, e0W2, e0b2), (e1W1, e1b1, e1W2, e1b2),
           (e2W1, e2b1, e2W2, e2b2))
    shared = (g0W.T, g0b.reshape(_C, 1), g1W.T, g1b.reshape(_C, 1),
              g2W.T, g2b.reshape(_C, 1), hW1.T, hb1.reshape(_C, 1),
              hW2.T, hb2.reshape(1, 1))
    outs = []
    for (H, W), f, (W1, b1, W2, b2) in zip(_LEVELS, feats, enc):
        N = H * W
        args = (f.reshape(_C, N), W1.T, b1.reshape(_C, 1), W2.T,
                b2.reshape(_C, 1)) + shared
        out = pl.pallas_call(
            functools.partial(_body, H, W),
            out_shape=jax.ShapeDtypeStruct((1, N), jnp.float32),
            scratch_shapes=[pltpu.VMEM((_C, N), jnp.float32),
                            pltpu.VMEM((_C, N), jnp.float32)],
        )(*args)
        outs.append(out.reshape(1, H, W, 1))
    return tuple(outs)


# speedup vs baseline: 19.2542x; 1.0353x over previous
import functools

import jax
import jax.numpy as jnp
from jax import lax
from jax.experimental import pallas as pl
from jax.experimental.pallas import tpu as pltpu

_C = 256
_LEVELS = [(128, 128), (64, 64), (32, 32)]
_CC = 32
_CH = 2048


def _body(H, W, f_ref, w1, b1, w2, b2, gw0, gb0, gw1, gb1, gw2, gb2,
          hw1, hb1, hw2, hb2, out_ref, A, B):
    N = H * W
    ch = min(_CH, N)
    logw = W.bit_length() - 1

    n = lax.broadcasted_iota(jnp.int32, (1, N), 1)
    col = n & (W - 1)
    row = lax.shift_right_logical(n, logw)
    f32 = jnp.float32
    top = (row == 0)
    bot = (row == H - 1)
    lft = (col == 0)
    rgt = (col == W - 1)
    deg = (5.0 - top.astype(f32) - bot.astype(f32)
           - lft.astype(f32) - rgt.astype(f32))
    dinv = lax.rsqrt(deg)
    mU = 1.0 - top.astype(f32)
    mD = 1.0 - bot.astype(f32)
    mL = 1.0 - lft.astype(f32)
    mR = 1.0 - rgt.astype(f32)

    def roll(x, k):
        return pltpu.roll(x, k % N, 1)

    bf = jnp.bfloat16

    for n0 in range(0, N, ch):
        sl = slice(n0, n0 + ch)
        x1 = jnp.maximum(
            jnp.dot(w1[:], f_ref[:, sl], preferred_element_type=f32) + b1[:],
            0.0)
        A[:, sl] = jnp.dot(w2[:], x1.astype(bf),
                           preferred_element_type=f32) + b2[:]

    layers = ((gw0, gb0, A, B), (gw1, gb1, B, A), (gw2, gb2, A, B))
    for gw, gb, src, dst in layers:
        for n0 in range(0, N, ch):
            sl = slice(n0, n0 + ch)
            src[:, sl] = jnp.dot(gw[:], (src[:, sl] * dinv[:, sl]).astype(bf),
                                 preferred_element_type=f32)
        for c0 in range(0, _C, _CC):
            cs = slice(c0, c0 + _CC)
            g = src[cs, :]
            agg = (g
                   + mU * roll(g, W) + mD * roll(g, -W)
                   + mL * roll(g, 1) + mR * roll(g, -1))
            dst[cs, :] = dinv * agg + gb[cs, :]

    for n0 in range(0, N, ch):
        sl = slice(n0, n0 + ch)
        t = jnp.maximum(
            jnp.dot(hw1[:], B[:, sl].astype(bf),
                    preferred_element_type=f32) + hb1[:],
            0.0)
        out_ref[:, sl] = jnp.dot(hw2[:], t.astype(bf),
                                 preferred_element_type=f32) + hb2[:]


@jax.jit
def kernel(feat0, feat1, feat2, e0W1, e0b1, e0W2, e0b2, e1W1, e1b1, e1W2,
           e1b2, e2W1, e2b1, e2W2, e2b2, g0W, g0b, g1W, g1b, g2W, g2b,
           hW1, hb1, hW2, hb2):
    feats = (feat0, feat1, feat2)
    enc = ((e0W1, e0b1, e0W2, e0b2), (e1W1, e1b1, e1W2, e1b2),
           (e2W1, e2b1, e2W2, e2b2))
    bf = jnp.bfloat16
    shared = (g0W.T.astype(bf), g0b.reshape(_C, 1), g1W.T.astype(bf),
              g1b.reshape(_C, 1), g2W.T.astype(bf), g2b.reshape(_C, 1),
              hW1.T.astype(bf), hb1.reshape(_C, 1), hW2.T.astype(bf),
              hb2.reshape(1, 1))
    outs = []
    for (H, W), f, (W1, b1, W2, b2) in zip(_LEVELS, feats, enc):
        N = H * W
        args = (f.reshape(_C, N).astype(bf), W1.T.astype(bf),
                b1.reshape(_C, 1), W2.T.astype(bf),
                b2.reshape(_C, 1)) + shared
        out = pl.pallas_call(
            functools.partial(_body, H, W),
            out_shape=jax.ShapeDtypeStruct((1, N), jnp.float32),
            scratch_shapes=[pltpu.VMEM((_C, N), jnp.float32),
                            pltpu.VMEM((_C, N), jnp.float32)],
        )(*args)
        outs.append(out.reshape(1, H, W, 1))
    return tuple(outs)
